# baseline (speedup 1.0000x reference)
import jax
import jax.numpy as jnp
from jax.experimental import pallas as pl
from jax.experimental.pallas import tpu as pltpu


def _wnn_kernel(x_ref, p0_ref, p1_ref, p2_ref, w1_ref, b1_ref, w2_ref, b2_ref,
                w_ref, ens_ref):
    xb = x_ref[...].astype(jnp.bfloat16)
    h = jax.lax.dot_general(
        w1_ref[...], xb, (((1,), (1,)), ((), ())),
        preferred_element_type=jnp.float32)
    h = jnp.maximum(h + b1_ref[...], 0.0).astype(jnp.bfloat16)
    raw = jnp.dot(w2_ref[...], h, preferred_element_type=jnp.float32)
    w8 = jax.nn.sigmoid(raw + b2_ref[...])
    w_ref[...] = w8
    ens_ref[...] = (w8[0:1] * p0_ref[...] + w8[1:2] * p1_ref[...]
                    + w8[2:3] * p2_ref[...])


def kernel(x_nn_scaled, target, pred_HBV, pred_PRMS, pred_SACSMA, W1, b1, W2, b2):
    T, B, NX = x_nn_scaled.shape
    Tt = target.shape[0]
    H = W1.shape[1]
    M = W2.shape[1]
    diff = T - Tt
    N = Tt * B
    OFF = diff * B

    C = 9344
    assert N % C == 0 and OFF % C == 0
    grid = N // C
    off_blocks = OFF // C

    MP = 8

    x2 = x_nn_scaled.reshape(T * B, NX)
    p0 = pred_HBV.reshape(1, N)
    p1 = pred_PRMS.reshape(1, N)
    p2 = pred_SACSMA.reshape(1, N)
    w1T = W1.T.astype(jnp.bfloat16)
    b1c = b1.reshape(H, 1)
    w2T = jnp.zeros((MP, H), jnp.bfloat16).at[:M].set(W2.T.astype(jnp.bfloat16))
    b2c = jnp.zeros((MP, 1), jnp.float32).at[:M, 0].set(b2)

    w8, ens = pl.pallas_call(
        _wnn_kernel,
        grid=(grid,),
        in_specs=[
            pl.BlockSpec((C, NX), lambda i: (i + off_blocks, 0)),
            pl.BlockSpec((1, C), lambda i: (0, i)),
            pl.BlockSpec((1, C), lambda i: (0, i)),
            pl.BlockSpec((1, C), lambda i: (0, i)),
            pl.BlockSpec((H, NX), lambda i: (0, 0)),
            pl.BlockSpec((H, 1), lambda i: (0, 0)),
            pl.BlockSpec((MP, H), lambda i: (0, 0)),
            pl.BlockSpec((MP, 1), lambda i: (0, 0)),
        ],
        out_specs=[
            pl.BlockSpec((MP, C), lambda i: (0, i)),
            pl.BlockSpec((1, C), lambda i: (0, i)),
        ],
        out_shape=[
            jax.ShapeDtypeStruct((MP, N), jnp.float32),
            jax.ShapeDtypeStruct((1, N), jnp.float32),
        ],
        compiler_params=pltpu.CompilerParams(
            dimension_semantics=("arbitrary",),
        ),
    )(x2, p0, p1, p2, w1T, b1c, w2T, b2c)

    ensemble = ens.reshape(Tt, B)
    w = w8[:M].reshape(M, Tt, B).transpose(1, 2, 0)
    return ensemble, w


# speedup vs baseline: 1.4630x; 1.4630x over previous
import jax
import jax.numpy as jnp
from jax.experimental import pallas as pl
from jax.experimental.pallas import tpu as pltpu


def _wnn_kernel(x_ref, p0_ref, p1_ref, p2_ref, w1_ref, b1_ref, w2_ref, b2_ref,
                w_ref, ens_ref):
    tb = x_ref.shape[0]
    H = w1_ref.shape[0]
    MP = w2_ref.shape[0]
    xb = x_ref[...].astype(jnp.bfloat16)
    w1b = jnp.broadcast_to(w1_ref[...], (tb, H, w1_ref.shape[1]))
    h = jax.lax.dot_general(
        w1b, xb, (((2,), (1,)), ((0,), (0,))),
        preferred_element_type=jnp.float32)
    h = jnp.maximum(h + b1_ref[...], 0.0).astype(jnp.bfloat16)
    w2b = jnp.broadcast_to(w2_ref[...], (tb, MP, H))
    raw = jax.lax.dot_general(
        w2b, h, (((2,), (1,)), ((0,), (0,))),
        preferred_element_type=jnp.float32)
    w8 = jax.nn.sigmoid(raw + b2_ref[...])
    w_ref[...] = w8
    ens_ref[...] = (w8[:, 0:1, :] * p0_ref[...] + w8[:, 1:2, :] * p1_ref[...]
                    + w8[:, 2:3, :] * p2_ref[...])


def kernel(x_nn_scaled, target, pred_HBV, pred_PRMS, pred_SACSMA, W1, b1, W2, b2):
    T, B, NX = x_nn_scaled.shape
    Tt = target.shape[0]
    H = W1.shape[1]
    M = W2.shape[1]
    diff = T - Tt

    TB = 73
    assert Tt % TB == 0 and diff % TB == 0
    grid = Tt // TB
    off_blocks = diff // TB

    MP = 8

    x3 = x_nn_scaled.transpose(0, 2, 1)
    p0 = pred_HBV.transpose(0, 2, 1)
    p1 = pred_PRMS.transpose(0, 2, 1)
    p2 = pred_SACSMA.transpose(0, 2, 1)
    w1T = W1.T.astype(jnp.bfloat16)
    b1c = b1.reshape(1, H, 1)
    w2T = jnp.zeros((MP, H), jnp.bfloat16).at[:M].set(W2.T.astype(jnp.bfloat16))
    b2c = jnp.zeros((1, MP, 1), jnp.float32).at[0, :M, 0].set(b2)

    w8, ens = pl.pallas_call(
        _wnn_kernel,
        grid=(grid,),
        in_specs=[
            pl.BlockSpec((TB, NX, B), lambda i: (i + off_blocks, 0, 0)),
            pl.BlockSpec((TB, 1, B), lambda i: (i, 0, 0)),
            pl.BlockSpec((TB, 1, B), lambda i: (i, 0, 0)),
            pl.BlockSpec((TB, 1, B), lambda i: (i, 0, 0)),
            pl.BlockSpec((H, NX), lambda i: (0, 0)),
            pl.BlockSpec((1, H, 1), lambda i: (0, 0, 0)),
            pl.BlockSpec((MP, H), lambda i: (0, 0)),
            pl.BlockSpec((1, MP, 1), lambda i: (0, 0, 0)),
        ],
        out_specs=[
            pl.BlockSpec((TB, MP, B), lambda i: (i, 0, 0)),
            pl.BlockSpec((TB, 1, B), lambda i: (i, 0, 0)),
        ],
        out_shape=[
            jax.ShapeDtypeStruct((Tt, MP, B), jnp.float32),
            jax.ShapeDtypeStruct((Tt, 1, B), jnp.float32),
        ],
        compiler_params=pltpu.CompilerParams(
            dimension_semantics=("arbitrary",),
        ),
    )(x3, p0, p1, p2, w1T, b1c, w2T, b2c)

    ensemble = ens.reshape(Tt, B)
    w = w8[:, :M, :].transpose(0, 2, 1)
    return ensemble, w


# speedup vs baseline: 1.8447x; 1.2609x over previous
import jax
import jax.numpy as jnp
from jax.experimental import pallas as pl
from jax.experimental.pallas import tpu as pltpu


def _wnn_kernel(x_ref, p0_ref, p1_ref, p2_ref, w1_ref, b1_ref, w2_ref, b2_ref,
                w_ref, ens_ref):
    TB = x_ref.shape[0]
    B = w_ref.shape[3]
    M = w_ref.shape[0]
    MP = w2_ref.shape[0]
    H, NX = w1_ref.shape
    xb = x_ref[...].astype(jnp.bfloat16)
    w1b = jnp.broadcast_to(w1_ref[...], (TB, H, NX))
    h = jax.lax.dot_general(
        w1b, xb, (((2,), (1,)), ((0,), (0,))),
        preferred_element_type=jnp.float32)
    h = jnp.maximum(h + b1_ref[...], 0.0).astype(jnp.bfloat16)
    w2b = jnp.broadcast_to(w2_ref[...], (TB, MP, H))
    raw = jax.lax.dot_general(
        w2b, h, (((2,), (1,)), ((0,), (0,))),
        preferred_element_type=jnp.float32)
    w8 = jax.nn.sigmoid(raw + b2_ref[...])
    w8t = jnp.transpose(w8, (1, 0, 2))
    w_ref[...] = w8t[:M].reshape(M, 1, TB, B)
    ens = (w8t[0:1] * p0_ref[...].reshape(1, TB, B)
           + w8t[1:2] * p1_ref[...].reshape(1, TB, B)
           + w8t[2:3] * p2_ref[...].reshape(1, TB, B))
    ens_ref[...] = ens


def kernel(x_nn_scaled, target, pred_HBV, pred_PRMS, pred_SACSMA, W1, b1, W2, b2):
    T, B, NX = x_nn_scaled.shape
    Tt = target.shape[0]
    H = W1.shape[1]
    M = W2.shape[1]
    diff = T - Tt

    TB = 73
    assert Tt % TB == 0 and diff % TB == 0
    grid = Tt // TB
    off_blocks = diff // TB

    MP = 8

    x3 = x_nn_scaled.transpose(0, 2, 1)
    p0 = pred_HBV.transpose(2, 0, 1).reshape(1, grid, TB, B)
    p1 = pred_PRMS.transpose(2, 0, 1).reshape(1, grid, TB, B)
    p2 = pred_SACSMA.transpose(2, 0, 1).reshape(1, grid, TB, B)
    w1T = W1.T.astype(jnp.bfloat16)
    b1c = b1.reshape(1, H, 1)
    w2T = jnp.zeros((MP, H), jnp.bfloat16).at[:M].set(W2.T.astype(jnp.bfloat16))
    b2c = jnp.zeros((1, MP, 1), jnp.float32).at[0, :M, 0].set(b2)

    w4, ens = pl.pallas_call(
        _wnn_kernel,
        grid=(grid,),
        in_specs=[
            pl.BlockSpec((TB, NX, B), lambda i: (i + off_blocks, 0, 0)),
            pl.BlockSpec((1, 1, TB, B), lambda i: (0, i, 0, 0)),
            pl.BlockSpec((1, 1, TB, B), lambda i: (0, i, 0, 0)),
            pl.BlockSpec((1, 1, TB, B), lambda i: (0, i, 0, 0)),
            pl.BlockSpec((H, NX), lambda i: (0, 0)),
            pl.BlockSpec((1, H, 1), lambda i: (0, 0, 0)),
            pl.BlockSpec((MP, H), lambda i: (0, 0)),
            pl.BlockSpec((1, MP, 1), lambda i: (0, 0, 0)),
        ],
        out_specs=[
            pl.BlockSpec((M, 1, TB, B), lambda i: (0, i, 0, 0)),
            pl.BlockSpec((1, TB, B), lambda i: (i, 0, 0)),
        ],
        out_shape=[
            jax.ShapeDtypeStruct((M, grid, TB, B), jnp.float32),
            jax.ShapeDtypeStruct((grid, TB, B), jnp.float32),
        ],
        compiler_params=pltpu.CompilerParams(
            dimension_semantics=("arbitrary",),
        ),
    )(x3, p0, p1, p2, w1T, b1c, w2T, b2c)

    w = w4.reshape(M, Tt, B).transpose(1, 2, 0)
    return ens.reshape(Tt, B), w


# speedup vs baseline: 1.9412x; 1.0523x over previous
import jax
import jax.numpy as jnp
from jax.experimental import pallas as pl
from jax.experimental.pallas import tpu as pltpu


def _wnn_kernel(x_ref, p0_ref, p1_ref, p2_ref, w1_ref, b1_ref, w2_ref, b2_ref,
                w_ref, ens_ref):
    TB = x_ref.shape[0]
    B = w_ref.shape[3]
    M = w_ref.shape[0]
    MP = w2_ref.shape[0]
    H, NX = w1_ref.shape
    xb = x_ref[...].astype(jnp.bfloat16)
    w1b = jnp.broadcast_to(w1_ref[...], (TB, H, NX))
    h = jax.lax.dot_general(
        w1b, xb, (((2,), (1,)), ((0,), (0,))),
        preferred_element_type=jnp.float32)
    h = jnp.maximum(h.astype(jnp.bfloat16) + b1_ref[...], 0)
    w2b = jnp.broadcast_to(w2_ref[...], (TB, MP, H))
    raw = jax.lax.dot_general(
        w2b, h, (((2,), (1,)), ((0,), (0,))),
        preferred_element_type=jnp.float32)
    w8 = jax.nn.sigmoid(raw + b2_ref[...])
    w8t = jnp.transpose(w8, (1, 0, 2))
    w_ref[...] = w8t[:M].reshape(M, 1, TB, B)
    ens_ref[...] = (w8[:, 0:1, :] * p0_ref[...] + w8[:, 1:2, :] * p1_ref[...]
                    + w8[:, 2:3, :] * p2_ref[...])


def kernel(x_nn_scaled, target, pred_HBV, pred_PRMS, pred_SACSMA, W1, b1, W2, b2):
    T, B, NX = x_nn_scaled.shape
    Tt = target.shape[0]
    H = W1.shape[1]
    M = W2.shape[1]
    diff = T - Tt

    TB = 73
    assert Tt % TB == 0 and diff % TB == 0
    grid = Tt // TB
    off_blocks = diff // TB

    MP = 8

    x3 = x_nn_scaled.transpose(0, 2, 1)
    p0 = pred_HBV.transpose(0, 2, 1)
    p1 = pred_PRMS.transpose(0, 2, 1)
    p2 = pred_SACSMA.transpose(0, 2, 1)
    w1T = W1.T.astype(jnp.bfloat16)
    b1c = b1.reshape(1, H, 1).astype(jnp.bfloat16)
    w2T = jnp.zeros((MP, H), jnp.bfloat16).at[:M].set(W2.T.astype(jnp.bfloat16))
    b2c = jnp.zeros((1, MP, 1), jnp.float32).at[0, :M, 0].set(b2)

    w4, ens = pl.pallas_call(
        _wnn_kernel,
        grid=(grid,),
        in_specs=[
            pl.BlockSpec((TB, NX, B), lambda i: (i + off_blocks, 0, 0)),
            pl.BlockSpec((TB, 1, B), lambda i: (i, 0, 0)),
            pl.BlockSpec((TB, 1, B), lambda i: (i, 0, 0)),
            pl.BlockSpec((TB, 1, B), lambda i: (i, 0, 0)),
            pl.BlockSpec((H, NX), lambda i: (0, 0)),
            pl.BlockSpec((1, H, 1), lambda i: (0, 0, 0)),
            pl.BlockSpec((MP, H), lambda i: (0, 0)),
            pl.BlockSpec((1, MP, 1), lambda i: (0, 0, 0)),
        ],
        out_specs=[
            pl.BlockSpec((M, 1, TB, B), lambda i: (0, i, 0, 0)),
            pl.BlockSpec((TB, 1, B), lambda i: (i, 0, 0)),
        ],
        out_shape=[
            jax.ShapeDtypeStruct((M, grid, TB, B), jnp.float32),
            jax.ShapeDtypeStruct((Tt, 1, B), jnp.float32),
        ],
        compiler_params=pltpu.CompilerParams(
            dimension_semantics=("arbitrary",),
        ),
    )(x3, p0, p1, p2, w1T, b1c, w2T, b2c)

    w = w4.reshape(M, Tt, B).transpose(1, 2, 0)
    return ens.reshape(Tt, B), w
